# baseline (speedup 1.0000x reference)
import functools

import jax
import jax.numpy as jnp
from jax import lax
from jax.experimental import pallas as pl
from jax.experimental.pallas import tpu as pltpu
from jax.experimental.pallas import tpu_sc as plsc

B = 16384
D = 32
PACK = 128 // D
NC = 2
NS = 16
NW = NC * NS
BPW = B // NW
CHUNK = 128
NCHUNK = BPW // CHUNK


def _sc_body(uq_h, uoff_h, mq_h, moff_h, ut_h, mt_h, wb_h, out_h,
             uq, uoff, mq, moff, urows, mrows, wbv, outv, sem):
    wid = lax.axis_index("s") * NC + lax.axis_index("c")
    base = wid * BPW

    pltpu.sync_copy(uq_h.at[pl.ds(base, BPW)], uq)
    pltpu.sync_copy(uoff_h.at[pl.ds(base, BPW)], uoff)
    pltpu.sync_copy(mq_h.at[pl.ds(base, BPW)], mq)
    pltpu.sync_copy(moff_h.at[pl.ds(base, BPW)], moff)
    pltpu.sync_copy(wb_h, wbv)

    wvecs = [wbv[pl.ds(i * 16, 16)] for i in range(5)]
    wu = [wvecs[k // 16][k % 16] for k in range(D)]
    wm = [wvecs[(D + k) // 16][(D + k) % 16] for k in range(D)]
    bias = wvecs[4][0]
    iota = lax.iota(jnp.int32, 16)

    for c in range(NCHUNK):
        sl = pl.ds(c * CHUNK, CHUNK)
        cp_u = pltpu.async_copy(ut_h.at[uq.at[sl]], urows, sem)
        cp_m = pltpu.async_copy(mt_h.at[mq.at[sl]], mrows, sem)
        cp_u.wait()
        cp_m.wait()

        def group(j, carry):
            row = iota + j * 16
            uo = uoff[pl.ds(c * CHUNK + j * 16, 16)]
            mo = moff[pl.ds(c * CHUNK + j * 16, 16)]
            acc = jnp.full((16,), bias, jnp.float32)
            for k in range(D):
                acc = acc + plsc.load_gather(urows, [row, uo + k]) * wu[k]
                acc = acc + plsc.load_gather(mrows, [row, mo + k]) * wm[k]
            outv[pl.ds(c * CHUNK + j * 16, 16)] = acc
            return carry

        lax.fori_loop(0, CHUNK // 16, group, 0)

    pltpu.sync_copy(outv, out_h.at[pl.ds(base, BPW)])


@functools.partial(jax.jit, static_argnames=())
def kernel(users, movies, user_table, movie_table, W, b):
    users = users.astype(jnp.int32)
    movies = movies.astype(jnp.int32)
    uq = users // PACK
    uoff = (users % PACK) * D
    mq = movies // PACK
    moff = (movies % PACK) * D
    ut2 = user_table.reshape(-1, PACK * D)
    mt2 = movie_table.reshape(-1, PACK * D)
    wb = jnp.concatenate(
        [W.reshape(-1), b.reshape(-1), jnp.zeros((15,), jnp.float32)])
    run = pl.kernel(
        _sc_body,
        mesh=plsc.VectorSubcoreMesh(core_axis_name="c", subcore_axis_name="s"),
        compiler_params=pltpu.CompilerParams(needs_layout_passes=False),
        out_type=jax.ShapeDtypeStruct((B,), jnp.float32),
        scratch_types=[
            pltpu.VMEM((BPW,), jnp.int32),
            pltpu.VMEM((BPW,), jnp.int32),
            pltpu.VMEM((BPW,), jnp.int32),
            pltpu.VMEM((BPW,), jnp.int32),
            pltpu.VMEM((CHUNK, PACK * D), jnp.float32),
            pltpu.VMEM((CHUNK, PACK * D), jnp.float32),
            pltpu.VMEM((2 * D + 16,), jnp.float32),
            pltpu.VMEM((BPW,), jnp.float32),
            pltpu.SemaphoreType.DMA,
        ],
    )
    out = run(uq, uoff, mq, moff, ut2, mt2, wb)
    return out.reshape(B, 1)


# speedup vs baseline: 1.5745x; 1.5745x over previous
import functools

import jax
import jax.numpy as jnp
from jax import lax
from jax.experimental import pallas as pl
from jax.experimental.pallas import tpu as pltpu
from jax.experimental.pallas import tpu_sc as plsc

B = 16384
D = 32
NC = 2
NS = 16
NW = NC * NS
BPW = B // NW
CHUNK = 64
NCHUNK = BPW // CHUNK


def _sc_body(users_h, movies_h, ut_h, mt_h, wb_h, out_h,
             uidx, midx, urows, mrows, wbv, outv, sem):
    wid = lax.axis_index("s") * NC + lax.axis_index("c")
    base = wid * BPW

    pltpu.sync_copy(users_h.at[pl.ds(base, BPW)], uidx)
    pltpu.sync_copy(movies_h.at[pl.ds(base, BPW)], midx)
    pltpu.sync_copy(wb_h, wbv)

    wvecs = [wbv[pl.ds(i * 16, 16)] for i in range(5)]
    wu = [wvecs[k // 16][k % 16] for k in range(D)]
    wm = [wvecs[(D + k) // 16][(D + k) % 16] for k in range(D)]
    bias = wvecs[4][0]
    iota = lax.iota(jnp.int32, 16)

    def chunk_body(c, carry):
        cbase = c * CHUNK
        copies = []
        for g in range(CHUNK // 16):
            uv = uidx[pl.ds(cbase + g * 16, 16)]
            mv = midx[pl.ds(cbase + g * 16, 16)]
            for l in range(16):
                i = g * 16 + l
                copies.append(pltpu.async_copy(
                    ut_h.at[pl.ds(uv[l], 1), :], urows.at[pl.ds(i, 1), :], sem))
                copies.append(pltpu.async_copy(
                    mt_h.at[pl.ds(mv[l], 1), :], mrows.at[pl.ds(i, 1), :], sem))
        for cp in copies:
            cp.wait()

        for j in range(CHUNK // 16):
            row = iota + j * 16
            acc = jnp.full((16,), bias, jnp.float32)
            for k in range(D):
                ck = jnp.full((16,), k, jnp.int32)
                acc = acc + plsc.load_gather(urows, [row, ck]) * wu[k]
                acc = acc + plsc.load_gather(mrows, [row, ck]) * wm[k]
            outv[pl.ds(cbase + j * 16, 16)] = acc
        return carry

    lax.fori_loop(0, NCHUNK, chunk_body, 0)
    pltpu.sync_copy(outv, out_h.at[pl.ds(base, BPW)])


@functools.partial(jax.jit, static_argnames=())
def kernel(users, movies, user_table, movie_table, W, b):
    users = users.astype(jnp.int32)
    movies = movies.astype(jnp.int32)
    wb = jnp.concatenate(
        [W.reshape(-1), b.reshape(-1), jnp.zeros((15,), jnp.float32)])
    run = pl.kernel(
        _sc_body,
        mesh=plsc.VectorSubcoreMesh(core_axis_name="c", subcore_axis_name="s"),
        compiler_params=pltpu.CompilerParams(needs_layout_passes=False),
        out_type=jax.ShapeDtypeStruct((B,), jnp.float32),
        scratch_types=[
            pltpu.VMEM((BPW,), jnp.int32),
            pltpu.VMEM((BPW,), jnp.int32),
            pltpu.VMEM((CHUNK, D), jnp.float32),
            pltpu.VMEM((CHUNK, D), jnp.float32),
            pltpu.VMEM((2 * D + 16,), jnp.float32),
            pltpu.VMEM((BPW,), jnp.float32),
            pltpu.SemaphoreType.DMA,
        ],
    )
    out = run(users, movies, user_table, movie_table, wb)
    return out.reshape(B, 1)


# speedup vs baseline: 1.6694x; 1.0603x over previous
import functools

import jax
import jax.numpy as jnp
from jax import lax
from jax.experimental import pallas as pl
from jax.experimental.pallas import tpu as pltpu
from jax.experimental.pallas import tpu_sc as plsc

B = 16384
D = 32
NC = 2
NS = 16
NW = NC * NS
BPW = B // NW
CHUNK = 128
NCHUNK = BPW // CHUNK

TCHUNK = 2048


def _matvec_body(w_ref, x_ref, o_ref):
    s = jnp.sum(x_ref[...] * w_ref[...], axis=0)
    o_ref[...] = s.reshape(TCHUNK // 128, 128)


def _scores(xT, w):
    n = xT.shape[1]
    grid = (n + TCHUNK - 1) // TCHUNK
    return pl.pallas_call(
        _matvec_body,
        grid=(grid,),
        in_specs=[
            pl.BlockSpec((D, 1), lambda n: (0, 0)),
            pl.BlockSpec((D, TCHUNK), lambda n: (0, n)),
        ],
        out_specs=pl.BlockSpec((TCHUNK // 128, 128), lambda n: (n, 0)),
        out_shape=jax.ShapeDtypeStruct((grid * TCHUNK // 128, 128), jnp.float32),
    )(w.reshape(D, 1), xT)


def _sc_body(uq_h, ulane_h, mq_h, mlane_h, su_h, sm_h, bias_h, out_h,
             uq, ulane, mq, mlane, ubuf, mbuf, bv, outv, sem):
    wid = lax.axis_index("s") * NC + lax.axis_index("c")
    base = wid * BPW

    pltpu.sync_copy(uq_h.at[pl.ds(base, BPW)], uq)
    pltpu.sync_copy(ulane_h.at[pl.ds(base, BPW)], ulane)
    pltpu.sync_copy(mq_h.at[pl.ds(base, BPW)], mq)
    pltpu.sync_copy(mlane_h.at[pl.ds(base, BPW)], mlane)
    pltpu.sync_copy(bias_h, bv)
    bias = bv[pl.ds(0, 16)][0]
    iota = lax.iota(jnp.int32, 16)

    for c in range(NCHUNK):
        sl = pl.ds(c * CHUNK, CHUNK)
        cp_u = pltpu.async_copy(su_h.at[uq.at[sl]], ubuf, sem)
        cp_m = pltpu.async_copy(sm_h.at[mq.at[sl]], mbuf, sem)
        cp_u.wait()
        cp_m.wait()

        def group(j, carry):
            row = iota + j * 16
            lu = ulane[pl.ds(c * CHUNK + j * 16, 16)]
            lm = mlane[pl.ds(c * CHUNK + j * 16, 16)]
            uval = plsc.load_gather(ubuf, [row, lu])
            mval = plsc.load_gather(mbuf, [row, lm])
            outv[pl.ds(c * CHUNK + j * 16, 16)] = uval + mval + bias
            return carry

        lax.fori_loop(0, CHUNK // 16, group, 0)

    pltpu.sync_copy(outv, out_h.at[pl.ds(base, BPW)])


@functools.partial(jax.jit, static_argnames=())
def kernel(users, movies, user_table, movie_table, W, b):
    users = users.astype(jnp.int32)
    movies = movies.astype(jnp.int32)
    uq = users >> 7
    ulane = users & 127
    mq = movies >> 7
    mlane = movies & 127
    su = _scores(user_table.T, W[0, :D])
    sm = _scores(movie_table.T, W[0, D:])
    bias = jnp.broadcast_to(b, (16,)).astype(jnp.float32)
    run = pl.kernel(
        _sc_body,
        mesh=plsc.VectorSubcoreMesh(core_axis_name="c", subcore_axis_name="s"),
        compiler_params=pltpu.CompilerParams(needs_layout_passes=False),
        out_type=jax.ShapeDtypeStruct((B,), jnp.float32),
        scratch_types=[
            pltpu.VMEM((BPW,), jnp.int32),
            pltpu.VMEM((BPW,), jnp.int32),
            pltpu.VMEM((BPW,), jnp.int32),
            pltpu.VMEM((BPW,), jnp.int32),
            pltpu.VMEM((CHUNK, 128), jnp.float32),
            pltpu.VMEM((CHUNK, 128), jnp.float32),
            pltpu.VMEM((16,), jnp.float32),
            pltpu.VMEM((BPW,), jnp.float32),
            pltpu.SemaphoreType.DMA,
        ],
    )
    out = run(uq, ulane, mq, mlane, su, sm, bias)
    return out.reshape(B, 1)


# speedup vs baseline: 5.4694x; 3.2763x over previous
import functools

import jax
import jax.numpy as jnp
from jax import lax
from jax.experimental import pallas as pl
from jax.experimental.pallas import tpu as pltpu
from jax.experimental.pallas import tpu_sc as plsc

B = 16384
D = 32
NC = 2
NS = 16
NW = NC * NS
BPW = B // NW
CHUNK = 128
NCHUNK = BPW // CHUNK

TCHUNK = 16384


def _matvec_body(w_ref, x_ref, o_ref):
    s = jax.lax.dot_general(
        w_ref[...], x_ref[...], (((1,), (0,)), ((), ())),
        preferred_element_type=jnp.float32)
    o_ref[...] = s.reshape(1, 1, TCHUNK)


def _scores(xT, w):
    n = xT.shape[1]
    grid = (n + TCHUNK - 1) // TCHUNK
    out = pl.pallas_call(
        _matvec_body,
        grid=(grid,),
        in_specs=[
            pl.BlockSpec((1, D), lambda n: (0, 0)),
            pl.BlockSpec((D, TCHUNK), lambda n: (0, n)),
        ],
        out_specs=pl.BlockSpec((1, 1, TCHUNK), lambda n: (n, 0, 0)),
        out_shape=jax.ShapeDtypeStruct((grid, 1, TCHUNK), jnp.float32),
    )(w.reshape(1, D), xT)
    return out.reshape(grid * TCHUNK // 128, 128)


def _sc_body(uq_h, ulane_h, mq_h, mlane_h, su_h, sm_h, bias_h, out_h,
             uq, ulane, mq, mlane, ubuf, mbuf, bv, outv, sem):
    wid = lax.axis_index("s") * NC + lax.axis_index("c")
    base = wid * BPW

    pltpu.sync_copy(uq_h.at[pl.ds(base, BPW)], uq)
    pltpu.sync_copy(ulane_h.at[pl.ds(base, BPW)], ulane)
    pltpu.sync_copy(mq_h.at[pl.ds(base, BPW)], mq)
    pltpu.sync_copy(mlane_h.at[pl.ds(base, BPW)], mlane)
    pltpu.sync_copy(bias_h, bv)
    bias = bv[pl.ds(0, 16)][0]
    iota = lax.iota(jnp.int32, 16)

    for c in range(NCHUNK):
        sl = pl.ds(c * CHUNK, CHUNK)
        cp_u = pltpu.async_copy(su_h.at[uq.at[sl]], ubuf, sem)
        cp_m = pltpu.async_copy(sm_h.at[mq.at[sl]], mbuf, sem)
        cp_u.wait()
        cp_m.wait()

        def group(j, carry):
            row = iota + j * 16
            lu = ulane[pl.ds(c * CHUNK + j * 16, 16)]
            lm = mlane[pl.ds(c * CHUNK + j * 16, 16)]
            uval = plsc.load_gather(ubuf, [row, lu])
            mval = plsc.load_gather(mbuf, [row, lm])
            outv[pl.ds(c * CHUNK + j * 16, 16)] = uval + mval + bias
            return carry

        lax.fori_loop(0, CHUNK // 16, group, 0)

    pltpu.sync_copy(outv, out_h.at[pl.ds(base, BPW)])


@functools.partial(jax.jit, static_argnames=())
def kernel(users, movies, user_table, movie_table, W, b):
    users = users.astype(jnp.int32)
    movies = movies.astype(jnp.int32)
    uq = users >> 7
    ulane = users & 127
    mq = movies >> 7
    mlane = movies & 127
    su = _scores(user_table.T, W[0, :D])
    sm = _scores(movie_table.T, W[0, D:])
    bias = jnp.broadcast_to(b, (16,)).astype(jnp.float32)
    run = pl.kernel(
        _sc_body,
        mesh=plsc.VectorSubcoreMesh(core_axis_name="c", subcore_axis_name="s"),
        compiler_params=pltpu.CompilerParams(needs_layout_passes=False),
        out_type=jax.ShapeDtypeStruct((B,), jnp.float32),
        scratch_types=[
            pltpu.VMEM((BPW,), jnp.int32),
            pltpu.VMEM((BPW,), jnp.int32),
            pltpu.VMEM((BPW,), jnp.int32),
            pltpu.VMEM((BPW,), jnp.int32),
            pltpu.VMEM((CHUNK, 128), jnp.float32),
            pltpu.VMEM((CHUNK, 128), jnp.float32),
            pltpu.VMEM((16,), jnp.float32),
            pltpu.VMEM((BPW,), jnp.float32),
            pltpu.SemaphoreType.DMA,
        ],
    )
    out = run(uq, ulane, mq, mlane, su, sm, bias)
    return out.reshape(B, 1)


# speedup vs baseline: 6.7345x; 1.2313x over previous
import functools

import jax
import jax.numpy as jnp
from jax import lax
from jax.experimental import pallas as pl
from jax.experimental.pallas import tpu as pltpu
from jax.experimental.pallas import tpu_sc as plsc

B = 16384
D = 32
NC = 2
NS = 16
NW = NC * NS
BPW = B // NW
CHUNK = 128
NCHUNK = BPW // CHUNK

TCHUNK = 32768


def _matvec_body(w_ref, x_ref, o_ref):
    s = jax.lax.dot_general(
        w_ref[...], x_ref[...], (((1,), (0,)), ((), ())),
        preferred_element_type=jnp.float32)
    o_ref[...] = s.reshape(1, 1, TCHUNK)


def _scores(xT, w):
    n = xT.shape[1]
    grid = (n + TCHUNK - 1) // TCHUNK
    out = pl.pallas_call(
        _matvec_body,
        grid=(grid,),
        in_specs=[
            pl.BlockSpec((1, D), lambda n: (0, 0)),
            pl.BlockSpec((D, TCHUNK), lambda n: (0, n)),
        ],
        out_specs=pl.BlockSpec((1, 1, TCHUNK), lambda n: (n, 0, 0)),
        out_shape=jax.ShapeDtypeStruct((grid, 1, TCHUNK), jnp.float32),
    )(w.reshape(1, D), xT)
    return out.reshape(grid * TCHUNK // 128, 128)


def _sc_body(uq_h, ulane_h, mq_h, mlane_h, su_h, sm_h, bias_h, out_h,
             uq, ulane, mq, mlane, ubuf, mbuf, bv, outv, sem):
    wid = lax.axis_index("s") * NC + lax.axis_index("c")
    base = wid * BPW

    pltpu.sync_copy(uq_h.at[pl.ds(base, BPW)], uq)
    pltpu.sync_copy(ulane_h.at[pl.ds(base, BPW)], ulane)
    pltpu.sync_copy(mq_h.at[pl.ds(base, BPW)], mq)
    pltpu.sync_copy(mlane_h.at[pl.ds(base, BPW)], mlane)
    pltpu.sync_copy(bias_h, bv)
    bias = bv[pl.ds(0, 16)][0]
    iota = lax.iota(jnp.int32, 16)

    for c in range(NCHUNK):
        sl = pl.ds(c * CHUNK, CHUNK)
        cp_u = pltpu.async_copy(su_h.at[uq.at[sl]], ubuf, sem)
        cp_m = pltpu.async_copy(sm_h.at[mq.at[sl]], mbuf, sem)
        cp_u.wait()
        cp_m.wait()

        def group(j, carry):
            row = iota + j * 16
            lu = ulane[pl.ds(c * CHUNK + j * 16, 16)]
            lm = mlane[pl.ds(c * CHUNK + j * 16, 16)]
            uval = plsc.load_gather(ubuf, [row, lu])
            mval = plsc.load_gather(mbuf, [row, lm])
            outv[pl.ds(c * CHUNK + j * 16, 16)] = uval + mval + bias
            return carry

        lax.fori_loop(0, CHUNK // 16, group, 0)

    pltpu.sync_copy(outv, out_h.at[pl.ds(base, BPW)])


@functools.partial(jax.jit, static_argnames=())
def kernel(users, movies, user_table, movie_table, W, b):
    users = users.astype(jnp.int32)
    movies = movies.astype(jnp.int32)
    uq = users >> 7
    ulane = users & 127
    mq = movies >> 7
    mlane = movies & 127
    su = _scores(user_table.T, W[0, :D])
    sm = _scores(movie_table.T, W[0, D:])
    bias = jnp.broadcast_to(b, (16,)).astype(jnp.float32)
    run = pl.kernel(
        _sc_body,
        mesh=plsc.VectorSubcoreMesh(core_axis_name="c", subcore_axis_name="s"),
        compiler_params=pltpu.CompilerParams(needs_layout_passes=False),
        out_type=jax.ShapeDtypeStruct((B,), jnp.float32),
        scratch_types=[
            pltpu.VMEM((BPW,), jnp.int32),
            pltpu.VMEM((BPW,), jnp.int32),
            pltpu.VMEM((BPW,), jnp.int32),
            pltpu.VMEM((BPW,), jnp.int32),
            pltpu.VMEM((CHUNK, 128), jnp.float32),
            pltpu.VMEM((CHUNK, 128), jnp.float32),
            pltpu.VMEM((16,), jnp.float32),
            pltpu.VMEM((BPW,), jnp.float32),
            pltpu.SemaphoreType.DMA,
        ],
    )
    out = run(uq, ulane, mq, mlane, su, sm, bias)
    return out.reshape(B, 1)


# speedup vs baseline: 7.1433x; 1.0607x over previous
import functools

import jax
import jax.numpy as jnp
from jax import lax
from jax.experimental import pallas as pl
from jax.experimental.pallas import tpu as pltpu
from jax.experimental.pallas import tpu_sc as plsc

B = 16384
D = 32
NC = 2
NS = 16
NW = NC * NS
BPW = B // NW
CHUNK = 128
NCHUNK = BPW // CHUNK

TCHUNK = 65536


def _matvec_body(w_ref, x_ref, o_ref):
    s = jax.lax.dot_general(
        w_ref[...], x_ref[...], (((1,), (0,)), ((), ())),
        preferred_element_type=jnp.float32)
    o_ref[...] = s.reshape(1, 1, TCHUNK)


def _scores(xT, w):
    n = xT.shape[1]
    grid = (n + TCHUNK - 1) // TCHUNK
    out = pl.pallas_call(
        _matvec_body,
        grid=(grid,),
        in_specs=[
            pl.BlockSpec((1, D), lambda n: (0, 0)),
            pl.BlockSpec((D, TCHUNK), lambda n: (0, n)),
        ],
        out_specs=pl.BlockSpec((1, 1, TCHUNK), lambda n: (n, 0, 0)),
        out_shape=jax.ShapeDtypeStruct((grid, 1, TCHUNK), jnp.float32),
    )(w.reshape(1, D), xT)
    return out.reshape(grid * TCHUNK // 128, 128)


def _sc_body(uq_h, ulane_h, mq_h, mlane_h, su_h, sm_h, bias_h, out_h,
             uq, ulane, mq, mlane, ubuf, mbuf, bv, outv, sem):
    wid = lax.axis_index("s") * NC + lax.axis_index("c")
    base = wid * BPW

    pltpu.sync_copy(uq_h.at[pl.ds(base, BPW)], uq)
    pltpu.sync_copy(ulane_h.at[pl.ds(base, BPW)], ulane)
    pltpu.sync_copy(mq_h.at[pl.ds(base, BPW)], mq)
    pltpu.sync_copy(mlane_h.at[pl.ds(base, BPW)], mlane)
    pltpu.sync_copy(bias_h, bv)
    bias = bv[pl.ds(0, 16)][0]
    iota = lax.iota(jnp.int32, 16)

    for c in range(NCHUNK):
        sl = pl.ds(c * CHUNK, CHUNK)
        cp_u = pltpu.async_copy(su_h.at[uq.at[sl]], ubuf, sem)
        cp_m = pltpu.async_copy(sm_h.at[mq.at[sl]], mbuf, sem)
        cp_u.wait()
        cp_m.wait()

        def group(j, carry):
            row = iota + j * 16
            lu = ulane[pl.ds(c * CHUNK + j * 16, 16)]
            lm = mlane[pl.ds(c * CHUNK + j * 16, 16)]
            uval = plsc.load_gather(ubuf, [row, lu])
            mval = plsc.load_gather(mbuf, [row, lm])
            outv[pl.ds(c * CHUNK + j * 16, 16)] = uval + mval + bias
            return carry

        lax.fori_loop(0, CHUNK // 16, group, 0)

    pltpu.sync_copy(outv, out_h.at[pl.ds(base, BPW)])


@functools.partial(jax.jit, static_argnames=())
def kernel(users, movies, user_table, movie_table, W, b):
    users = users.astype(jnp.int32)
    movies = movies.astype(jnp.int32)
    uq = users >> 7
    ulane = users & 127
    mq = movies >> 7
    mlane = movies & 127
    su = _scores(user_table.T, W[0, :D])
    sm = _scores(movie_table.T, W[0, D:])
    bias = jnp.broadcast_to(b, (16,)).astype(jnp.float32)
    run = pl.kernel(
        _sc_body,
        mesh=plsc.VectorSubcoreMesh(core_axis_name="c", subcore_axis_name="s"),
        compiler_params=pltpu.CompilerParams(needs_layout_passes=False),
        out_type=jax.ShapeDtypeStruct((B,), jnp.float32),
        scratch_types=[
            pltpu.VMEM((BPW,), jnp.int32),
            pltpu.VMEM((BPW,), jnp.int32),
            pltpu.VMEM((BPW,), jnp.int32),
            pltpu.VMEM((BPW,), jnp.int32),
            pltpu.VMEM((CHUNK, 128), jnp.float32),
            pltpu.VMEM((CHUNK, 128), jnp.float32),
            pltpu.VMEM((16,), jnp.float32),
            pltpu.VMEM((BPW,), jnp.float32),
            pltpu.SemaphoreType.DMA,
        ],
    )
    out = run(uq, ulane, mq, mlane, su, sm, bias)
    return out.reshape(B, 1)
